# Initial kernel scaffold; baseline (speedup 1.0000x reference)
import jax
import jax.numpy as jnp
from jax.experimental import pallas as pl


def kernel(x, edge_index, W, a_src, a_dst, b):
    raise NotImplementedError("write your pallas kernel here")


# speedup vs baseline: 27.9654x; 27.9654x over previous
import functools

import jax
import jax.numpy as jnp
from jax import lax
from jax.experimental import pallas as pl
from jax.experimental.pallas import tpu as pltpu
from jax.experimental.pallas import tpu_sc as plsc

N = 10000
E = 320000
F = 128
C = 128

NPAD = 10240
NCORES = 2
NSUB = 16
NW = NCORES * NSUB
EW = E // NW
CH = 128
NFULL = EW // CH
TAIL = EW - NFULL * CH


def _tc_a_body(x_ref, w_ref, a2_ref, h_ref, al_ref):
    h = jnp.dot(x_ref[...], w_ref[...], preferred_element_type=jnp.float32)
    h_ref[...] = h
    al_ref[...] = lax.dot_general(
        a2_ref[...], h,
        dimension_numbers=(((0,), (1,)), ((), ())),
        preferred_element_type=jnp.float32)


def _tc_a(xp, W, A2):
    BM = 1024
    grid = (NPAD // BM,)
    return pl.pallas_call(
        _tc_a_body,
        grid=grid,
        in_specs=[
            pl.BlockSpec((BM, F), lambda i: (i, 0)),
            pl.BlockSpec((F, C), lambda i: (0, 0)),
            pl.BlockSpec((C, 8), lambda i: (0, 0)),
        ],
        out_specs=[
            pl.BlockSpec((BM, C), lambda i: (i, 0)),
            pl.BlockSpec((8, BM), lambda i: (0, i)),
        ],
        out_shape=[
            jax.ShapeDtypeStruct((NPAD, C), jnp.float32),
            jax.ShapeDtypeStruct((8, NPAD), jnp.float32),
        ],
    )(xp, W, A2)


def _edge_chunk(K, base_e, src_hbm, dst_hbm, h_hbm, sidx_v, didx_v, w_v, rows_v,
                asrc_v, adst_v, acc_sh, den_sh, sem):
    pltpu.sync_copy(src_hbm.at[pl.ds(base_e, K)], sidx_v)
    pltpu.sync_copy(dst_hbm.at[pl.ds(base_e, K)], didx_v)

    def alpha_body(j, _):
        si = sidx_v[pl.ds(j * 16, 16)]
        di = didx_v[pl.ds(j * 16, 16)]
        z = plsc.load_gather(asrc_v, [si]) + plsc.load_gather(adst_v, [di])
        z = jnp.maximum(z, 0.2 * z)
        w_v[pl.ds(j * 16, 16)] = jnp.exp(z)
        return _
    lax.fori_loop(0, K // 16, alpha_body, None)

    pltpu.sync_copy(w_v, den_sh.at[didx_v], add=True)

    pltpu.async_copy(h_hbm.at[sidx_v], rows_v, sem).wait()

    def scale_body(jo, _):
        wvec = w_v[pl.ds(jo * 16, 16)]
        for lane in range(16):
            wb = lax.broadcast(wvec[lane], (16,))
            j = jo * 16 + lane
            for v in range(C // 16):
                sl = pl.ds(v * 16, 16)
                rows_v[j, sl] = rows_v[j, sl] * wb
        return _
    lax.fori_loop(0, K // 16, scale_body, None)

    pltpu.sync_copy(rows_v, acc_sh.at[didx_v], add=True)


def _sc_body(h_hbm, src_hbm, dst_hbm, asrc_hbm, adst_hbm, acc_out, den_out,
             asrc_v, adst_v, sidx_v, didx_v, w_v, rows_v,
             sidx_t, didx_t, w_t, rows_t, acc_sh, den_sh, sem):
    c = lax.axis_index("c")
    s = lax.axis_index("s")
    wid = c * NSUB + s
    rbase = s * (NPAD // NSUB)

    def zero_body(j, _):
        zv = jnp.zeros((16,), jnp.float32)
        for v in range(C // 16):
            rows_v[j, pl.ds(v * 16, 16)] = zv
        return _
    lax.fori_loop(0, CH, zero_body, None)
    for v in range(CH // 16):
        w_v[pl.ds(v * 16, 16)] = jnp.zeros((16,), jnp.float32)
    for k in range(5):
        pltpu.sync_copy(rows_v, acc_sh.at[pl.ds(rbase + k * CH, CH), :])
        pltpu.sync_copy(w_v, den_sh.at[pl.ds(rbase + k * CH, CH)])

    pltpu.sync_copy(asrc_hbm, asrc_v)
    pltpu.sync_copy(adst_hbm, adst_v)

    plsc.subcore_barrier()

    def chunk_body(ch, _):
        _edge_chunk(CH, wid * EW + ch * CH, src_hbm, dst_hbm, h_hbm, sidx_v,
                    didx_v, w_v, rows_v, asrc_v, adst_v, acc_sh, den_sh, sem)
        return _
    lax.fori_loop(0, NFULL, chunk_body, None)
    _edge_chunk(TAIL, wid * EW + NFULL * CH, src_hbm, dst_hbm, h_hbm, sidx_t,
                didx_t, w_t, rows_t, asrc_v, adst_v, acc_sh, den_sh, sem)

    plsc.subcore_barrier()

    nrt = NPAD // NSUB
    pltpu.sync_copy(acc_sh.at[pl.ds(rbase, nrt), :], acc_out.at[c, pl.ds(rbase, nrt), :])
    pltpu.sync_copy(den_sh.at[pl.ds(rbase, nrt)], den_out.at[pl.ds(c * NPAD + rbase, nrt)])


def _sc_edges(h, src, dst, asrc, adst):
    mesh = plsc.VectorSubcoreMesh(core_axis_name="c", subcore_axis_name="s")
    f = pl.kernel(
        _sc_body,
        out_type=[
            jax.ShapeDtypeStruct((NCORES, NPAD, C), jnp.float32),
            jax.ShapeDtypeStruct((NCORES * NPAD,), jnp.float32),
        ],
        mesh=mesh,
        scratch_types=[
            pltpu.VMEM((N,), jnp.float32),
            pltpu.VMEM((N,), jnp.float32),
            pltpu.VMEM((CH,), jnp.int32),
            pltpu.VMEM((CH,), jnp.int32),
            pltpu.VMEM((CH,), jnp.float32),
            pltpu.VMEM((CH, C), jnp.float32),
            pltpu.VMEM((TAIL,), jnp.int32),
            pltpu.VMEM((TAIL,), jnp.int32),
            pltpu.VMEM((TAIL,), jnp.float32),
            pltpu.VMEM((TAIL, C), jnp.float32),
            pltpu.VMEM_SHARED((NPAD, C), jnp.float32),
            pltpu.VMEM_SHARED((NPAD,), jnp.float32),
            pltpu.SemaphoreType.DMA,
        ],
        compiler_params=pltpu.CompilerParams(needs_layout_passes=False),
    )
    return f(h, src, dst, asrc, adst)


def _tc_b_body(acc_ref, den_ref, b_ref, o_ref):
    a = acc_ref[0] + acc_ref[1]
    d = den_ref[0] + den_ref[1] + 1e-9
    o = a / d + b_ref[...]
    o_ref[...] = jnp.maximum(o, 0.3 * o)


def _tc_b(acc, den3, b2):
    BM = 1024
    grid = (NPAD // BM,)
    return pl.pallas_call(
        _tc_b_body,
        grid=grid,
        in_specs=[
            pl.BlockSpec((NCORES, BM, C), lambda i: (0, i, 0)),
            pl.BlockSpec((NCORES, BM, 1), lambda i: (0, i, 0)),
            pl.BlockSpec((1, C), lambda i: (0, 0)),
        ],
        out_specs=pl.BlockSpec((BM, C), lambda i: (i, 0)),
        out_shape=jax.ShapeDtypeStruct((NPAD, C), jnp.float32),
    )(acc, den3, b2)


def kernel(x, edge_index, W, a_src, a_dst, b):
    ei = edge_index.astype(jnp.int32)
    A2 = jnp.zeros((C, 8), jnp.float32).at[:, 0].set(a_src).at[:, 1].set(a_dst)
    xp = jnp.pad(x, ((0, NPAD - N), (0, 0)))
    h, alT = _tc_a(xp, W, A2)
    acc, den = _sc_edges(h, ei[0], ei[1], alT[0, :N], alT[1, :N])
    out = _tc_b(acc, den.reshape(NCORES, NPAD, 1), b.reshape(1, C))
    return out[:N]


# speedup vs baseline: 54.8553x; 1.9615x over previous
import jax
import jax.numpy as jnp
from jax import lax
from jax.experimental import pallas as pl
from jax.experimental.pallas import tpu as pltpu
from jax.experimental.pallas import tpu_sc as plsc

N = 10000
E = 320000
F = 128
C = 128

NPAD = 10240
NCORES = 2
NSUB = 16
NW = NCORES * NSUB
EW = E // NW
CH = 128
NFULL = EW // CH
TAIL = EW - NFULL * CH


def _tc_a_body(x_ref, w_ref, a2_ref, h_ref, al_ref):
    h = jnp.dot(x_ref[...], w_ref[...], preferred_element_type=jnp.float32)
    h_ref[...] = h
    al_ref[0] = lax.dot_general(
        a2_ref[...], h,
        dimension_numbers=(((0,), (1,)), ((), ())),
        preferred_element_type=jnp.float32)


def _tc_a(x, W, A2):
    BM = 1000
    grid = (N // BM,)
    return pl.pallas_call(
        _tc_a_body,
        grid=grid,
        in_specs=[
            pl.BlockSpec((BM, F), lambda i: (i, 0)),
            pl.BlockSpec((F, C), lambda i: (0, 0)),
            pl.BlockSpec((C, 8), lambda i: (0, 0)),
        ],
        out_specs=[
            pl.BlockSpec((BM, C), lambda i: (i, 0)),
            pl.BlockSpec((1, 8, BM), lambda i: (i, 0, 0)),
        ],
        out_shape=[
            jax.ShapeDtypeStruct((N, C), jnp.float32),
            jax.ShapeDtypeStruct((N // BM, 8, BM), jnp.float32),
        ],
    )(x, W, A2)


def _sc_body(h_hbm, ei_hbm, asrc_hbm, adst_hbm, acc_out, den_out,
             sidx4, didx4, w4, asv4, adv4, zbuf,
             sidx_t, didx_t, w_t, asv_t, adv_t, rows_t, rows2,
             acc_sh, den_sh,
             si0, si1, si2, si3, sd0, sd1, sd2, sd3, sg0, sg1, sc0, sc1):
    c = lax.axis_index("c")
    s = lax.axis_index("s")
    wid = c * NSUB + s
    ebase = wid * EW
    rbase = s * (NPAD // NSUB)
    semi = (si0, si1, si2, si3)
    semd = (sd0, sd1, sd2, sd3)
    semg = (sg0, sg1)
    sems = (sc0, sc1)

    def zero_body(j, carry):
        zv = jnp.zeros((16,), jnp.float32)
        for v in range(C // 16):
            rows2[0, j, pl.ds(v * 16, 16)] = zv
        return carry
    lax.fori_loop(0, CH, zero_body, None)
    for v in range(CH // 16):
        zbuf[pl.ds(v * 16, 16)] = jnp.zeros((16,), jnp.float32)
    for k in range(5):
        pltpu.sync_copy(rows2.at[0], acc_sh.at[pl.ds(rbase + k * CH, CH), :])
        pltpu.sync_copy(zbuf, den_sh.at[pl.ds(rbase + k * CH, CH)])

    plsc.subcore_barrier()

    def stage_I(ch, q):
        pltpu.async_copy(ei_hbm.at[pl.ds(ebase + ch * CH, CH)], sidx4.at[q], semi[q])
        pltpu.async_copy(ei_hbm.at[pl.ds(E + ebase + ch * CH, CH)], didx4.at[q], semi[q])

    def stage_A(ch, q):
        pltpu.make_async_copy(ei_hbm.at[pl.ds(ebase + ch * CH, CH)], sidx4.at[q], semi[q]).wait()
        pltpu.make_async_copy(ei_hbm.at[pl.ds(E + ebase + ch * CH, CH)], didx4.at[q], semi[q]).wait()
        pltpu.async_copy(asrc_hbm.at[sidx4.at[q]], asv4.at[q], semi[q])
        pltpu.async_copy(adst_hbm.at[didx4.at[q]], adv4.at[q], semi[q])

    def stage_Wc(ch, q):
        pltpu.make_async_copy(asrc_hbm.at[sidx4.at[q]], asv4.at[q], semi[q]).wait()
        pltpu.make_async_copy(adst_hbm.at[didx4.at[q]], adv4.at[q], semi[q]).wait()

        def wb_(j, carry):
            sl = pl.ds(j * 16, 16)
            z = asv4[q, sl] + adv4[q, sl]
            z = jnp.maximum(z, 0.2 * z)
            w4[q, sl] = jnp.exp(z)
            return carry
        lax.fori_loop(0, CH // 16, wb_, None)
        pltpu.async_copy(w4.at[q], den_sh.at[didx4.at[q]], semd[q], add=True)

    def wait_den(q):
        pltpu.make_async_copy(w4.at[q], den_sh.at[didx4.at[q]], semd[q]).wait()

    def stage_G(ch, q, p):
        pltpu.async_copy(h_hbm.at[sidx4.at[q]], rows2.at[p], semg[p])

    def stage_B(ch, q, p):
        pltpu.make_async_copy(h_hbm.at[sidx4.at[q]], rows2.at[p], semg[p]).wait()

        def sb(jo, carry):
            wvec = w4[q, pl.ds(jo * 16, 16)]
            for lane in range(16):
                wb = lax.broadcast(wvec[lane], (16,))
                j = jo * 16 + lane
                for v in range(C // 16):
                    sl = pl.ds(v * 16, 16)
                    rows2[p, j, sl] = rows2[p, j, sl] * wb
            return carry
        lax.fori_loop(0, CH // 16, sb, None)
        pltpu.async_copy(rows2.at[p], acc_sh.at[didx4.at[q]], sems[p], add=True)

    def wait_scat(q, p):
        pltpu.make_async_copy(rows2.at[p], acc_sh.at[didx4.at[q]], sems[p]).wait()

    stage_I(0, 0)
    stage_I(1, 1)
    stage_A(0, 0)
    stage_I(2, 2); stage_A(1, 1); stage_Wc(0, 0); stage_G(0, 0, 0)
    stage_I(3, 3); stage_A(2, 2); stage_Wc(1, 1); stage_G(1, 1, 1); stage_B(0, 0, 0)
    wait_den(0); wait_scat(0, 0)
    stage_I(4, 0); stage_A(3, 3); stage_Wc(2, 2); stage_G(2, 2, 0); stage_B(1, 1, 1)
    wait_den(1); wait_scat(1, 1)
    stage_I(5, 1); stage_A(4, 0); stage_Wc(3, 3); stage_G(3, 3, 1); stage_B(2, 2, 0)

    def quad(i, carry):
        for k in range(4):
            t = 4 + i * 4 + k
            wait_den((k + 2) % 4)
            wait_scat((k + 2) % 4, k % 2)
            stage_I(t + 2, (k + 2) % 4)
            stage_A(t + 1, (k + 1) % 4)
            stage_Wc(t, k)
            stage_G(t, k, k % 2)
            stage_B(t - 1, (k + 3) % 4, (k + 1) % 2)
        return carry
    lax.fori_loop(0, (NFULL - 6) // 4, quad, None)

    wait_den(2); wait_scat(2, 0)
    stage_A(NFULL - 1, 1)
    stage_Wc(NFULL - 2, 0); stage_G(NFULL - 2, 0, 0); stage_B(NFULL - 3, 3, 1)
    wait_den(3); wait_scat(3, 1)
    stage_Wc(NFULL - 1, 1); stage_G(NFULL - 1, 1, 1); stage_B(NFULL - 2, 0, 0)
    stage_B(NFULL - 1, 1, 1)

    d1 = pltpu.async_copy(ei_hbm.at[pl.ds(ebase + NFULL * CH, TAIL)], sidx_t, si0)
    d2 = pltpu.async_copy(ei_hbm.at[pl.ds(E + ebase + NFULL * CH, TAIL)], didx_t, si1)
    d1.wait()
    d2.wait()
    d3 = pltpu.async_copy(asrc_hbm.at[sidx_t], asv_t, si0)
    d4 = pltpu.async_copy(adst_hbm.at[didx_t], adv_t, si1)
    d3.wait()
    d4.wait()
    z = asv_t[...] + adv_t[...]
    z = jnp.maximum(z, 0.2 * z)
    w_t[...] = jnp.exp(z)
    pltpu.sync_copy(w_t, den_sh.at[didx_t], add=True)
    pltpu.async_copy(h_hbm.at[sidx_t], rows_t, si0).wait()
    wvec = w_t[...]
    for lane in range(TAIL):
        wb = lax.broadcast(wvec[lane], (16,))
        for v in range(C // 16):
            sl = pl.ds(v * 16, 16)
            rows_t[lane, sl] = rows_t[lane, sl] * wb
    pltpu.sync_copy(rows_t, acc_sh.at[didx_t], add=True)

    wait_den(0); wait_den(1)
    wait_scat(0, 0); wait_scat(1, 1)

    plsc.subcore_barrier()

    nrt = NPAD // NSUB
    pltpu.sync_copy(acc_sh.at[pl.ds(rbase, nrt), :], acc_out.at[c, pl.ds(rbase, nrt), :])
    pltpu.sync_copy(den_sh.at[pl.ds(rbase, nrt)], den_out.at[pl.ds(c * NPAD + rbase, nrt)])


def _sc_edges(h, eiflat, asrc, adst):
    mesh = plsc.VectorSubcoreMesh(core_axis_name="c", subcore_axis_name="s")
    f = pl.kernel(
        _sc_body,
        out_type=[
            jax.ShapeDtypeStruct((NCORES, NPAD, C), jnp.float32),
            jax.ShapeDtypeStruct((NCORES * NPAD,), jnp.float32),
        ],
        mesh=mesh,
        scratch_types=[
            pltpu.VMEM((4, CH), jnp.int32),
            pltpu.VMEM((4, CH), jnp.int32),
            pltpu.VMEM((4, CH), jnp.float32),
            pltpu.VMEM((4, CH), jnp.float32),
            pltpu.VMEM((4, CH), jnp.float32),
            pltpu.VMEM((CH,), jnp.float32),
            pltpu.VMEM((TAIL,), jnp.int32),
            pltpu.VMEM((TAIL,), jnp.int32),
            pltpu.VMEM((TAIL,), jnp.float32),
            pltpu.VMEM((TAIL,), jnp.float32),
            pltpu.VMEM((TAIL,), jnp.float32),
            pltpu.VMEM((TAIL, C), jnp.float32),
            pltpu.VMEM((2, CH, C), jnp.float32),
            pltpu.VMEM_SHARED((NPAD, C), jnp.float32),
            pltpu.VMEM_SHARED((NPAD,), jnp.float32),
        ] + [pltpu.SemaphoreType.DMA] * 12,
        compiler_params=pltpu.CompilerParams(needs_layout_passes=False),
    )
    return f(h, eiflat, asrc, adst)


def _tc_b_body(acc_ref, den_ref, b_ref, o_ref):
    a = acc_ref[0] + acc_ref[1]
    d = den_ref[0] + den_ref[1] + 1e-9
    o = a / d + b_ref[...]
    o_ref[...] = jnp.maximum(o, 0.3 * o)


def _tc_b(acc, den3, b2):
    BM = 1000
    grid = (N // BM,)
    return pl.pallas_call(
        _tc_b_body,
        grid=grid,
        in_specs=[
            pl.BlockSpec((NCORES, BM, C), lambda i: (0, i, 0)),
            pl.BlockSpec((NCORES, BM, 1), lambda i: (0, i, 0)),
            pl.BlockSpec((1, C), lambda i: (0, 0)),
        ],
        out_specs=pl.BlockSpec((BM, C), lambda i: (i, 0)),
        out_shape=jax.ShapeDtypeStruct((N, C), jnp.float32),
    )(acc, den3, b2)


def kernel(x, edge_index, W, a_src, a_dst, b):
    eiflat = edge_index.astype(jnp.int32).reshape(2 * E)
    A2 = jnp.zeros((C, 8), jnp.float32).at[:, 0].set(a_src).at[:, 1].set(a_dst)
    h, alT = _tc_a(x, W, A2)
    asrc = alT[:, 0, :].reshape(N)
    adst = alT[:, 1, :].reshape(N)
    acc, den = _sc_edges(h, eiflat, asrc, adst)
    out = _tc_b(acc, den.reshape(NCORES, NPAD, 1), b.reshape(1, C))
    return out
